# baseline (speedup 1.0000x reference)
import functools

import jax
import jax.numpy as jnp
from jax import lax
from jax.experimental import pallas as pl
from jax.experimental.pallas import tpu as pltpu
from jax.experimental.pallas import tpu_sc as plsc

B = 4
N = 2048
K = 16
CI = 128
CO = 256
C3 = 3 * CO
NBLK_A = 256
NBLK_C = 512
SLABS = C3 // K
NWORKERS = 32
SLABS_PER_W = SLABS * B // NWORKERS


def _stage_a_body(p1_ref, p2t_ref, p2_ref, x1_ref, s2_ref,
                  wgs_ref, wgd_ref, wax_ref, wad_ref, ba_ref,
                  idx_ref, g_ref, a_ref):
    p1 = p1_ref[0]
    p2t = p2t_ref[0]

    dx = p1[:, 0:1] - p2t[0:1, :]
    dy = p1[:, 1:2] - p2t[1:2, :]
    dz = p1[:, 2:3] - p2t[2:3, :]
    d = dx * dx + dy * dy + dz * dz

    col = lax.broadcasted_iota(jnp.int32, d.shape, 1)
    vals = d
    cols = []
    for _ in range(K):
        m = jnp.min(vals, axis=1, keepdims=True)
        cand = jnp.where(vals == m, col, N * 2)
        arg = jnp.min(cand, axis=1, keepdims=True)
        cols.append(arg)
        vals = jnp.where(col == arg, jnp.float32(jnp.inf), vals)
    idx_ref[0] = jnp.concatenate(cols, axis=1)

    hi = jax.lax.Precision.HIGHEST
    s2 = s2_ref[0]
    p2 = p2_ref[0]
    g = lax.dot_general(s2, wgs_ref[...], (((0,), (1,)), ((), ())),
                        precision=hi)
    g = g + lax.dot_general(p2, wgd_ref[...], (((1,), (1,)), ((), ())),
                            precision=hi)
    g_ref[0] = g

    x1 = x1_ref[0]
    a = lax.dot_general(x1, wax_ref[...], (((0,), (1,)), ((), ())),
                        precision=hi)
    a = a + lax.dot_general(p1, wad_ref[...], (((1,), (1,)), ((), ())),
                            precision=hi)
    a_ref[0] = a + ba_ref[...]


NFOLD = N // 8


def _gather_max_body(g_hbm, idx_hbm, out_hbm, idx_v, g_v, o_v):
    cid = lax.axis_index("c")
    sid = lax.axis_index("s")
    wid = sid * 2 + cid
    b = wid // (NWORKERS // B)
    t = wid % (NWORKERS // B)
    pltpu.sync_copy(idx_hbm.at[b], idx_v)

    for gi in range(SLABS_PER_W):
        g = t * SLABS_PER_W + gi
        pltpu.sync_copy(g_hbm.at[b, g], g_v)

        def body(n, carry):
            iv = idx_v[n >> 3, pl.ds((n & 7) * K, K)]
            m = iv[0]
            acc = g_v[m >> 3, pl.ds((m & 7) * K, K)]
            for k in range(1, K):
                m = iv[k]
                acc = jnp.maximum(acc, g_v[m >> 3, pl.ds((m & 7) * K, K)])
            o_v[n >> 3, pl.ds((n & 7) * K, K)] = acc
            return carry

        lax.fori_loop(0, N, body, 0)
        pltpu.sync_copy(o_v, out_hbm.at[b, g])


def _stage_c_body(pre_ref, a_ref, x1_ref, wfc_ref, bfc_ref, out_ref):
    hi = jax.lax.Precision.HIGHEST
    t = pre_ref[0] + a_ref[0]
    zn = 1.0 / (1.0 + jnp.exp(-t[:, :CO]))
    rn = 1.0 / (1.0 + jnp.exp(-t[:, CO:2 * CO]))
    sold_n = t[:, 2 * CO:]
    rs = rn * sold_n
    x1 = x1_ref[0]
    snew = lax.dot_general(wfc_ref[:, :CI], x1, (((1,), (0,)), ((), ())),
                           precision=hi)
    snew = snew + lax.dot_general(wfc_ref[:, CI:], rs,
                                  (((1,), (1,)), ((), ())), precision=hi)
    snew = jnp.tanh(snew + bfc_ref[...])
    zc = zn.T
    soldc = sold_n.T
    out_ref[0] = zc * soldc + (1.0 - zc) * snew


def kernel(P1, X1, P2, S2, Wz, bz, Wr, br, Ws, bs, Wfc, bfc):
    f32 = jnp.float32
    WGs = jnp.concatenate([Wz[:, :CO], Wr[:, :CO], Ws[:, :CO]], 0)
    WGd = jnp.concatenate([Wz[:, CO + CI:], Wr[:, CO + CI:], Ws[:, CO:]], 0)
    WAx = jnp.concatenate([Wz[:, CO:CO + CI], Wr[:, CO:CO + CI],
                           jnp.zeros((CO, CI), f32)], 0)
    WAd = -WGd
    BA = jnp.concatenate([bz, br, bs]).reshape(1, C3)
    P2T = jnp.transpose(P2, (0, 2, 1))

    grid_a = (B, N // NBLK_A)
    idx_g, G, A = pl.pallas_call(
        _stage_a_body,
        grid=grid_a,
        in_specs=[
            pl.BlockSpec((1, NBLK_A, 3), lambda b, i: (b, i, 0)),
            pl.BlockSpec((1, 3, N), lambda b, i: (b, 0, 0)),
            pl.BlockSpec((1, NBLK_A, 3), lambda b, i: (b, i, 0)),
            pl.BlockSpec((1, CI, NBLK_A), lambda b, i: (b, 0, i)),
            pl.BlockSpec((1, CO, NBLK_A), lambda b, i: (b, 0, i)),
            pl.BlockSpec((C3, CO), lambda b, i: (0, 0)),
            pl.BlockSpec((C3, 3), lambda b, i: (0, 0)),
            pl.BlockSpec((C3, CI), lambda b, i: (0, 0)),
            pl.BlockSpec((C3, 3), lambda b, i: (0, 0)),
            pl.BlockSpec((1, C3), lambda b, i: (0, 0)),
        ],
        out_specs=[
            pl.BlockSpec((1, NBLK_A, K), lambda b, i: (b, i, 0)),
            pl.BlockSpec((1, NBLK_A, C3), lambda b, i: (b, i, 0)),
            pl.BlockSpec((1, NBLK_A, C3), lambda b, i: (b, i, 0)),
        ],
        out_shape=[
            jax.ShapeDtypeStruct((B, N, K), jnp.int32),
            jax.ShapeDtypeStruct((B, N, C3), f32),
            jax.ShapeDtypeStruct((B, N, C3), f32),
        ],
    )(P1, P2T, P2, X1, S2, WGs, WGd, WAx, WAd, BA)

    G_sc = (G.reshape(B, NFOLD, 8, SLABS, K)
             .transpose(0, 3, 1, 2, 4)
             .reshape(B, SLABS, NFOLD, 128))
    mesh = plsc.VectorSubcoreMesh(core_axis_name="c", subcore_axis_name="s")
    pre_sc = pl.kernel(
        _gather_max_body,
        out_type=jax.ShapeDtypeStruct((B, SLABS, NFOLD, 128), f32),
        mesh=mesh,
        scratch_types=[
            pltpu.VMEM((NFOLD, 128), jnp.int32),
            pltpu.VMEM((NFOLD, 128), f32),
            pltpu.VMEM((NFOLD, 128), f32),
        ],
    )(G_sc, idx_g.reshape(B, NFOLD, 128))
    pre = (pre_sc.reshape(B, SLABS, NFOLD, 8, K)
                 .transpose(0, 2, 3, 1, 4)
                 .reshape(B, N, C3))

    grid_c = (B, N // NBLK_C)
    S1 = pl.pallas_call(
        _stage_c_body,
        grid=grid_c,
        in_specs=[
            pl.BlockSpec((1, NBLK_C, C3), lambda b, i: (b, i, 0)),
            pl.BlockSpec((1, NBLK_C, C3), lambda b, i: (b, i, 0)),
            pl.BlockSpec((1, CI, NBLK_C), lambda b, i: (b, 0, i)),
            pl.BlockSpec((CO, CO + CI), lambda b, i: (0, 0)),
            pl.BlockSpec((CO, 1), lambda b, i: (0, 0)),
        ],
        out_specs=pl.BlockSpec((1, CO, NBLK_C), lambda b, i: (b, 0, i)),
        out_shape=jax.ShapeDtypeStruct((B, CO, N), f32),
    )(pre, A, X1, Wfc, bfc.reshape(CO, 1))

    return (P1, S1)


# speedup vs baseline: 1.1860x; 1.1860x over previous
import functools

import jax
import jax.numpy as jnp
from jax import lax
from jax.experimental import pallas as pl
from jax.experimental.pallas import tpu as pltpu
from jax.experimental.pallas import tpu_sc as plsc

B = 4
N = 2048
K = 16
CI = 128
CO = 256
C3 = 3 * CO
NBLK_A = 256
NBLK_C = 512
SLABS = C3 // K
NWORKERS = 32
SLABS_PER_W = SLABS * B // NWORKERS


def _stage_a_body(p1_ref, p2t_ref, p2_ref, x1_ref, s2_ref,
                  wgs_ref, wgd_ref, wax_ref, wad_ref, ba_ref,
                  idx_ref, g_ref, a_ref):
    p1 = p1_ref[0]
    p2t = p2t_ref[0]

    dx = p1[:, 0:1] - p2t[0:1, :]
    dy = p1[:, 1:2] - p2t[1:2, :]
    dz = p1[:, 2:3] - p2t[2:3, :]
    d = dx * dx + dy * dy + dz * dz

    col = lax.broadcasted_iota(jnp.int32, d.shape, 1)
    vals = d
    cols = []
    for _ in range(K):
        m = jnp.min(vals, axis=1, keepdims=True)
        cand = jnp.where(vals == m, col, N * 2)
        arg = jnp.min(cand, axis=1, keepdims=True)
        cols.append(arg)
        vals = jnp.where(col == arg, jnp.float32(jnp.inf), vals)
    idx_ref[0] = jnp.concatenate(cols, axis=1)

    hi = jax.lax.Precision.HIGHEST
    s2 = s2_ref[0]
    p2 = p2_ref[0]
    g = lax.dot_general(s2, wgs_ref[...], (((0,), (1,)), ((), ())),
                        precision=hi)
    g = g + lax.dot_general(p2, wgd_ref[...], (((1,), (1,)), ((), ())),
                            precision=hi)
    gf = g.reshape(NBLK_A // 8, 8, SLABS, K).transpose(2, 0, 1, 3)
    g_ref[0] = gf.reshape(SLABS, NBLK_A // 8, 128)

    x1 = x1_ref[0]
    a = lax.dot_general(x1, wax_ref[...], (((0,), (1,)), ((), ())),
                        precision=hi)
    a = a + lax.dot_general(p1, wad_ref[...], (((1,), (1,)), ((), ())),
                            precision=hi)
    a_ref[0] = a + ba_ref[...]


NFOLD = N // 8


def _gather_max_body(g_hbm, idx_hbm, out_hbm, idx_v, g_v, o_v):
    cid = lax.axis_index("c")
    sid = lax.axis_index("s")
    wid = sid * 2 + cid
    b = wid // (NWORKERS // B)
    t = wid % (NWORKERS // B)
    pltpu.sync_copy(idx_hbm.at[b], idx_v)

    for gi in range(SLABS_PER_W):
        g = t * SLABS_PER_W + gi
        pltpu.sync_copy(g_hbm.at[b, g], g_v)

        def body(n, carry):
            iv = idx_v[n >> 3, pl.ds((n & 7) * K, K)]
            m = iv[0]
            acc = g_v[m >> 3, pl.ds((m & 7) * K, K)]
            for k in range(1, K):
                m = iv[k]
                acc = jnp.maximum(acc, g_v[m >> 3, pl.ds((m & 7) * K, K)])
            o_v[n >> 3, pl.ds((n & 7) * K, K)] = acc
            return carry

        lax.fori_loop(0, N, body, 0)
        pltpu.sync_copy(o_v, out_hbm.at[b, g])


def _stage_c_body(pre_ref, a_ref, x1_ref, wfc_ref, bfc_ref, out_ref):
    hi = jax.lax.Precision.HIGHEST
    pf = pre_ref[0].reshape(SLABS, NBLK_C // 8, 8, K).transpose(1, 2, 0, 3)
    t = pf.reshape(NBLK_C, C3) + a_ref[0]
    zn = 1.0 / (1.0 + jnp.exp(-t[:, :CO]))
    rn = 1.0 / (1.0 + jnp.exp(-t[:, CO:2 * CO]))
    sold_n = t[:, 2 * CO:]
    rs = rn * sold_n
    x1 = x1_ref[0]
    snew = lax.dot_general(wfc_ref[:, :CI], x1, (((1,), (0,)), ((), ())),
                           precision=hi)
    snew = snew + lax.dot_general(wfc_ref[:, CI:], rs,
                                  (((1,), (1,)), ((), ())), precision=hi)
    snew = jnp.tanh(snew + bfc_ref[...])
    zc = zn.T
    soldc = sold_n.T
    out_ref[0] = zc * soldc + (1.0 - zc) * snew


def kernel(P1, X1, P2, S2, Wz, bz, Wr, br, Ws, bs, Wfc, bfc):
    f32 = jnp.float32
    WGs = jnp.concatenate([Wz[:, :CO], Wr[:, :CO], Ws[:, :CO]], 0)
    WGd = jnp.concatenate([Wz[:, CO + CI:], Wr[:, CO + CI:], Ws[:, CO:]], 0)
    WAx = jnp.concatenate([Wz[:, CO:CO + CI], Wr[:, CO:CO + CI],
                           jnp.zeros((CO, CI), f32)], 0)
    WAd = -WGd
    BA = jnp.concatenate([bz, br, bs]).reshape(1, C3)
    P2T = jnp.transpose(P2, (0, 2, 1))

    grid_a = (B, N // NBLK_A)
    idx_g, G, A = pl.pallas_call(
        _stage_a_body,
        grid=grid_a,
        in_specs=[
            pl.BlockSpec((1, NBLK_A, 3), lambda b, i: (b, i, 0)),
            pl.BlockSpec((1, 3, N), lambda b, i: (b, 0, 0)),
            pl.BlockSpec((1, NBLK_A, 3), lambda b, i: (b, i, 0)),
            pl.BlockSpec((1, CI, NBLK_A), lambda b, i: (b, 0, i)),
            pl.BlockSpec((1, CO, NBLK_A), lambda b, i: (b, 0, i)),
            pl.BlockSpec((C3, CO), lambda b, i: (0, 0)),
            pl.BlockSpec((C3, 3), lambda b, i: (0, 0)),
            pl.BlockSpec((C3, CI), lambda b, i: (0, 0)),
            pl.BlockSpec((C3, 3), lambda b, i: (0, 0)),
            pl.BlockSpec((1, C3), lambda b, i: (0, 0)),
        ],
        out_specs=[
            pl.BlockSpec((1, NBLK_A, K), lambda b, i: (b, i, 0)),
            pl.BlockSpec((1, SLABS, NBLK_A // 8, 128), lambda b, i: (b, 0, i, 0)),
            pl.BlockSpec((1, NBLK_A, C3), lambda b, i: (b, i, 0)),
        ],
        out_shape=[
            jax.ShapeDtypeStruct((B, N, K), jnp.int32),
            jax.ShapeDtypeStruct((B, SLABS, NFOLD, 128), f32),
            jax.ShapeDtypeStruct((B, N, C3), f32),
        ],
    )(P1, P2T, P2, X1, S2, WGs, WGd, WAx, WAd, BA)

    G_sc = G
    mesh = plsc.VectorSubcoreMesh(core_axis_name="c", subcore_axis_name="s")
    pre_sc = pl.kernel(
        _gather_max_body,
        out_type=jax.ShapeDtypeStruct((B, SLABS, NFOLD, 128), f32),
        mesh=mesh,
        scratch_types=[
            pltpu.VMEM((NFOLD, 128), jnp.int32),
            pltpu.VMEM((NFOLD, 128), f32),
            pltpu.VMEM((NFOLD, 128), f32),
        ],
    )(G_sc, idx_g.reshape(B, NFOLD, 128))

    grid_c = (B, N // NBLK_C)
    S1 = pl.pallas_call(
        _stage_c_body,
        grid=grid_c,
        in_specs=[
            pl.BlockSpec((1, SLABS, NBLK_C // 8, 128), lambda b, i: (b, 0, i, 0)),
            pl.BlockSpec((1, NBLK_C, C3), lambda b, i: (b, i, 0)),
            pl.BlockSpec((1, CI, NBLK_C), lambda b, i: (b, 0, i)),
            pl.BlockSpec((CO, CO + CI), lambda b, i: (0, 0)),
            pl.BlockSpec((CO, 1), lambda b, i: (0, 0)),
        ],
        out_specs=pl.BlockSpec((1, CO, NBLK_C), lambda b, i: (b, 0, i)),
        out_shape=jax.ShapeDtypeStruct((B, CO, N), f32),
    )(pre_sc, A, X1, Wfc, bfc.reshape(CO, 1))

    return (P1, S1)


# speedup vs baseline: 1.4349x; 1.2098x over previous
import functools

import jax
import jax.numpy as jnp
from jax import lax
from jax.experimental import pallas as pl
from jax.experimental.pallas import tpu as pltpu
from jax.experimental.pallas import tpu_sc as plsc

B = 4
N = 2048
K = 16
CI = 128
CO = 256
C3 = 3 * CO
NBLK_A = 256
NBLK_C = 512
SLABS = C3 // K
NWORKERS = 32
SLABS_PER_W = SLABS * B // NWORKERS


def _stage_a_body(p1_ref, p2t_ref, p2_ref, x1_ref, s2_ref,
                  wgs_ref, wgd_ref, wax_ref, wad_ref, ba_ref,
                  idx_ref, g_ref, a_ref):
    p1 = p1_ref[0]
    p2t = p2t_ref[0]

    dx = p1[:, 0:1] - p2t[0:1, :]
    dy = p1[:, 1:2] - p2t[1:2, :]
    dz = p1[:, 2:3] - p2t[2:3, :]
    d = dx * dx + dy * dy + dz * dz

    col = lax.broadcasted_iota(jnp.int32, d.shape, 1)
    vals = d
    cols = []
    for _ in range(K):
        m = jnp.min(vals, axis=1, keepdims=True)
        cand = jnp.where(vals == m, col, N * 2)
        arg = jnp.min(cand, axis=1, keepdims=True)
        cols.append(arg)
        vals = jnp.where(col == arg, jnp.float32(jnp.inf), vals)
    idx_ref[0] = jnp.concatenate(cols, axis=1)

    hi = jax.lax.Precision.HIGHEST
    s2 = s2_ref[0]
    p2 = p2_ref[0]
    g = lax.dot_general(s2, wgs_ref[...], (((0,), (1,)), ((), ())),
                        precision=hi)
    g = g + lax.dot_general(p2, wgd_ref[...], (((1,), (1,)), ((), ())),
                            precision=hi)
    gf = g.reshape(NBLK_A // 8, 8, SLABS, K).transpose(2, 0, 1, 3)
    g_ref[0] = gf.reshape(SLABS, NBLK_A // 8, 128)

    x1 = x1_ref[0]
    a = lax.dot_general(x1, wax_ref[...], (((0,), (1,)), ((), ())),
                        precision=hi)
    a = a + lax.dot_general(p1, wad_ref[...], (((1,), (1,)), ((), ())),
                            precision=hi)
    a_ref[0] = a + ba_ref[...]


NFOLD = N // 8


NHALF = N // 2
UNITS_PER_W = 2 * SLABS // NWORKERS


def _gather_max_body(g_hbm, idx_hbm, out_hbm, idx_v, g_v, o_v):
    cid = lax.axis_index("c")
    sid = lax.axis_index("s")
    wid = sid * 2 + cid

    for j in range(UNITS_PER_W):
        u = wid + NWORKERS * j
        g = u >> 1
        half = u & 1
        pltpu.sync_copy(idx_hbm.at[0, pl.ds(half * (NHALF // 8), NHALF // 8)],
                        idx_v)
        pltpu.sync_copy(g_hbm.at[0, g], g_v)

        def body(n, carry):
            iv = idx_v[n >> 3, pl.ds((n & 7) * K, K)]
            m = iv[0]
            acc = g_v[m >> 3, pl.ds((m & 7) * K, K)]
            for k in range(1, K):
                m = iv[k]
                acc = jnp.maximum(acc, g_v[m >> 3, pl.ds((m & 7) * K, K)])
            o_v[n >> 3, pl.ds((n & 7) * K, K)] = acc
            return carry

        lax.fori_loop(0, NHALF, body, 0)
        pltpu.sync_copy(
            o_v, out_hbm.at[0, g, pl.ds(half * (NHALF // 8), NHALF // 8)])


def _stage_c_body(pre_ref, a_ref, x1_ref, wfc_ref, bfc_ref, out_ref):
    hi = jax.lax.Precision.HIGHEST
    pf = pre_ref[0].reshape(SLABS, NBLK_C // 8, 8, K).transpose(1, 2, 0, 3)
    t = pf.reshape(NBLK_C, C3) + a_ref[0]
    zn = 1.0 / (1.0 + jnp.exp(-t[:, :CO]))
    rn = 1.0 / (1.0 + jnp.exp(-t[:, CO:2 * CO]))
    sold_n = t[:, 2 * CO:]
    rs = rn * sold_n
    x1 = x1_ref[0]
    snew = lax.dot_general(wfc_ref[:, :CI], x1, (((1,), (0,)), ((), ())),
                           precision=hi)
    snew = snew + lax.dot_general(wfc_ref[:, CI:], rs,
                                  (((1,), (1,)), ((), ())), precision=hi)
    snew = jnp.tanh(snew + bfc_ref[...])
    zc = zn.T
    soldc = sold_n.T
    out_ref[0] = zc * soldc + (1.0 - zc) * snew


def kernel(P1, X1, P2, S2, Wz, bz, Wr, br, Ws, bs, Wfc, bfc):
    f32 = jnp.float32
    WGs = jnp.concatenate([Wz[:, :CO], Wr[:, :CO], Ws[:, :CO]], 0)
    WGd = jnp.concatenate([Wz[:, CO + CI:], Wr[:, CO + CI:], Ws[:, CO:]], 0)
    WAx = jnp.concatenate([Wz[:, CO:CO + CI], Wr[:, CO:CO + CI],
                           jnp.zeros((CO, CI), f32)], 0)
    WAd = -WGd
    BA = jnp.concatenate([bz, br, bs]).reshape(1, C3)
    P2T = jnp.transpose(P2, (0, 2, 1))

    mesh = plsc.VectorSubcoreMesh(core_axis_name="c", subcore_axis_name="s")
    bfc2 = bfc.reshape(CO, 1)
    s1_parts = []
    for b in range(B):
        idx_b, G_b, A_b = pl.pallas_call(
            _stage_a_body,
            grid=(N // NBLK_A,),
            in_specs=[
                pl.BlockSpec((1, NBLK_A, 3), lambda i, b=b: (b, i, 0)),
                pl.BlockSpec((1, 3, N), lambda i, b=b: (b, 0, 0)),
                pl.BlockSpec((1, NBLK_A, 3), lambda i, b=b: (b, i, 0)),
                pl.BlockSpec((1, CI, NBLK_A), lambda i, b=b: (b, 0, i)),
                pl.BlockSpec((1, CO, NBLK_A), lambda i, b=b: (b, 0, i)),
                pl.BlockSpec((C3, CO), lambda i: (0, 0)),
                pl.BlockSpec((C3, 3), lambda i: (0, 0)),
                pl.BlockSpec((C3, CI), lambda i: (0, 0)),
                pl.BlockSpec((C3, 3), lambda i: (0, 0)),
                pl.BlockSpec((1, C3), lambda i: (0, 0)),
            ],
            out_specs=[
                pl.BlockSpec((1, NBLK_A, K), lambda i: (0, i, 0)),
                pl.BlockSpec((1, SLABS, NBLK_A // 8, 128), lambda i: (0, 0, i, 0)),
                pl.BlockSpec((1, NBLK_A, C3), lambda i: (0, i, 0)),
            ],
            out_shape=[
                jax.ShapeDtypeStruct((1, N, K), jnp.int32),
                jax.ShapeDtypeStruct((1, SLABS, NFOLD, 128), f32),
                jax.ShapeDtypeStruct((1, N, C3), f32),
            ],
        )(P1, P2T, P2, X1, S2, WGs, WGd, WAx, WAd, BA)

        pre_b = pl.kernel(
            _gather_max_body,
            out_type=jax.ShapeDtypeStruct((1, SLABS, NFOLD, 128), f32),
            mesh=mesh,
            scratch_types=[
                pltpu.VMEM((NFOLD // 2, 128), jnp.int32),
                pltpu.VMEM((NFOLD, 128), f32),
                pltpu.VMEM((NFOLD // 2, 128), f32),
            ],
        )(G_b, idx_b.reshape(1, NFOLD, 128))

        s1_b = pl.pallas_call(
            _stage_c_body,
            grid=(N // NBLK_C,),
            in_specs=[
                pl.BlockSpec((1, SLABS, NBLK_C // 8, 128), lambda i: (0, 0, i, 0)),
                pl.BlockSpec((1, NBLK_C, C3), lambda i: (0, i, 0)),
                pl.BlockSpec((1, CI, NBLK_C), lambda i, b=b: (b, 0, i)),
                pl.BlockSpec((CO, CO + CI), lambda i: (0, 0)),
                pl.BlockSpec((CO, 1), lambda i: (0, 0)),
            ],
            out_specs=pl.BlockSpec((1, CO, NBLK_C), lambda i: (0, 0, i)),
            out_shape=jax.ShapeDtypeStruct((1, CO, N), f32),
        )(pre_b, A_b, X1, Wfc, bfc2)
        s1_parts.append(s1_b)

    S1 = jnp.concatenate(s1_parts, axis=0)
    return (P1, S1)


# speedup vs baseline: 1.6096x; 1.1217x over previous
import functools

import jax
import jax.numpy as jnp
from jax import lax
from jax.experimental import pallas as pl
from jax.experimental.pallas import tpu as pltpu
from jax.experimental.pallas import tpu_sc as plsc

B = 4
N = 2048
K = 16
CI = 128
CO = 256
C3 = 3 * CO
NBLK_A = 256
NBLK_C = 512
SLABS = C3 // K
NWORKERS = 32
SLABS_PER_W = SLABS * B // NWORKERS


def _stage_a_body(p1_ref, p2t_ref, p2_ref, x1_ref, s2_ref,
                  wgs_ref, wgd_ref, wax_ref, wad_ref, ba_ref,
                  idx_ref, g_ref, a_ref):
    p1 = p1_ref[0]
    p2t = p2t_ref[0]

    dx = p1[:, 0:1] - p2t[0:1, :]
    dy = p1[:, 1:2] - p2t[1:2, :]
    dz = p1[:, 2:3] - p2t[2:3, :]
    d = dx * dx + dy * dy + dz * dz

    colf = lax.broadcasted_iota(jnp.int32, d.shape, 1).astype(jnp.float32)
    vals = d
    cols = []
    for _ in range(K):
        m = jnp.min(vals, axis=1, keepdims=True)
        cand = jnp.where(vals == m, colf, jnp.float32(2 * N))
        arg = jnp.min(cand, axis=1, keepdims=True)
        cols.append(arg)
        vals = jnp.where(colf == arg, jnp.float32(jnp.inf), vals)
    idx_ref[0] = jnp.concatenate(cols, axis=1).astype(jnp.int32)

    hi = jax.lax.Precision.HIGHEST
    s2 = s2_ref[0]
    p2 = p2_ref[0]
    g = lax.dot_general(s2, wgs_ref[...], (((0,), (1,)), ((), ())),
                        precision=hi)
    g = g + lax.dot_general(p2, wgd_ref[...], (((1,), (1,)), ((), ())),
                            precision=hi)
    gf = g.reshape(NBLK_A // 8, 8, SLABS, K).transpose(2, 0, 1, 3)
    g_ref[0] = gf.reshape(SLABS, NBLK_A // 8, 128)

    x1 = x1_ref[0]
    a = lax.dot_general(x1, wax_ref[...], (((0,), (1,)), ((), ())),
                        precision=hi)
    a = a + lax.dot_general(p1, wad_ref[...], (((1,), (1,)), ((), ())),
                            precision=hi)
    a_ref[0] = a + ba_ref[...]


NFOLD = N // 8


NHALF = N // 2
UNITS_PER_W = 2 * SLABS // NWORKERS


def _gather_max_body(g_hbm, idx_hbm, out_hbm, idx_v, g_v, o_v):
    cid = lax.axis_index("c")
    sid = lax.axis_index("s")
    wid = sid * 2 + cid

    for j in range(UNITS_PER_W):
        u = wid + NWORKERS * j
        g = u >> 1
        half = u & 1
        pltpu.sync_copy(idx_hbm.at[0, pl.ds(half * (NHALF // 8), NHALF // 8)],
                        idx_v)
        pltpu.sync_copy(g_hbm.at[0, g], g_v)

        def body(n, carry):
            iv = idx_v[n >> 3, pl.ds((n & 7) * K, K)]
            m = iv[0]
            acc = g_v[m >> 3, pl.ds((m & 7) * K, K)]
            for k in range(1, K):
                m = iv[k]
                acc = jnp.maximum(acc, g_v[m >> 3, pl.ds((m & 7) * K, K)])
            o_v[n >> 3, pl.ds((n & 7) * K, K)] = acc
            return carry

        lax.fori_loop(0, NHALF, body, 0)
        pltpu.sync_copy(
            o_v, out_hbm.at[0, g, pl.ds(half * (NHALF // 8), NHALF // 8)])


def _stage_c_body(pre_ref, a_ref, x1_ref, wfc_ref, bfc_ref, out_ref):
    hi = jax.lax.Precision.HIGHEST
    pf = pre_ref[0].reshape(SLABS, NBLK_C // 8, 8, K).transpose(1, 2, 0, 3)
    t = pf.reshape(NBLK_C, C3) + a_ref[0]
    zn = 1.0 / (1.0 + jnp.exp(-t[:, :CO]))
    rn = 1.0 / (1.0 + jnp.exp(-t[:, CO:2 * CO]))
    sold_n = t[:, 2 * CO:]
    rs = rn * sold_n
    x1 = x1_ref[0]
    snew = lax.dot_general(wfc_ref[:, :CI], x1, (((1,), (0,)), ((), ())),
                           precision=hi)
    snew = snew + lax.dot_general(wfc_ref[:, CI:], rs,
                                  (((1,), (1,)), ((), ())), precision=hi)
    snew = jnp.tanh(snew + bfc_ref[...])
    zc = zn.T
    soldc = sold_n.T
    out_ref[0] = zc * soldc + (1.0 - zc) * snew


def kernel(P1, X1, P2, S2, Wz, bz, Wr, br, Ws, bs, Wfc, bfc):
    f32 = jnp.float32
    WGs = jnp.concatenate([Wz[:, :CO], Wr[:, :CO], Ws[:, :CO]], 0)
    WGd = jnp.concatenate([Wz[:, CO + CI:], Wr[:, CO + CI:], Ws[:, CO:]], 0)
    WAx = jnp.concatenate([Wz[:, CO:CO + CI], Wr[:, CO:CO + CI],
                           jnp.zeros((CO, CI), f32)], 0)
    WAd = -WGd
    BA = jnp.concatenate([bz, br, bs]).reshape(1, C3)
    P2T = jnp.transpose(P2, (0, 2, 1))

    mesh = plsc.VectorSubcoreMesh(core_axis_name="c", subcore_axis_name="s")
    bfc2 = bfc.reshape(CO, 1)
    s1_parts = []
    for b in range(B):
        idx_b, G_b, A_b = pl.pallas_call(
            _stage_a_body,
            grid=(N // NBLK_A,),
            in_specs=[
                pl.BlockSpec((1, NBLK_A, 3), lambda i, b=b: (b, i, 0)),
                pl.BlockSpec((1, 3, N), lambda i, b=b: (b, 0, 0)),
                pl.BlockSpec((1, NBLK_A, 3), lambda i, b=b: (b, i, 0)),
                pl.BlockSpec((1, CI, NBLK_A), lambda i, b=b: (b, 0, i)),
                pl.BlockSpec((1, CO, NBLK_A), lambda i, b=b: (b, 0, i)),
                pl.BlockSpec((C3, CO), lambda i: (0, 0)),
                pl.BlockSpec((C3, 3), lambda i: (0, 0)),
                pl.BlockSpec((C3, CI), lambda i: (0, 0)),
                pl.BlockSpec((C3, 3), lambda i: (0, 0)),
                pl.BlockSpec((1, C3), lambda i: (0, 0)),
            ],
            out_specs=[
                pl.BlockSpec((1, NBLK_A, K), lambda i: (0, i, 0)),
                pl.BlockSpec((1, SLABS, NBLK_A // 8, 128), lambda i: (0, 0, i, 0)),
                pl.BlockSpec((1, NBLK_A, C3), lambda i: (0, i, 0)),
            ],
            out_shape=[
                jax.ShapeDtypeStruct((1, N, K), jnp.int32),
                jax.ShapeDtypeStruct((1, SLABS, NFOLD, 128), f32),
                jax.ShapeDtypeStruct((1, N, C3), f32),
            ],
        )(P1, P2T, P2, X1, S2, WGs, WGd, WAx, WAd, BA)

        pre_b = pl.kernel(
            _gather_max_body,
            out_type=jax.ShapeDtypeStruct((1, SLABS, NFOLD, 128), f32),
            mesh=mesh,
            scratch_types=[
                pltpu.VMEM((NFOLD // 2, 128), jnp.int32),
                pltpu.VMEM((NFOLD, 128), f32),
                pltpu.VMEM((NFOLD // 2, 128), f32),
            ],
        )(G_b, idx_b.reshape(1, NFOLD, 128))

        s1_b = pl.pallas_call(
            _stage_c_body,
            grid=(N // NBLK_C,),
            in_specs=[
                pl.BlockSpec((1, SLABS, NBLK_C // 8, 128), lambda i: (0, 0, i, 0)),
                pl.BlockSpec((1, NBLK_C, C3), lambda i: (0, i, 0)),
                pl.BlockSpec((1, CI, NBLK_C), lambda i, b=b: (b, 0, i)),
                pl.BlockSpec((CO, CO + CI), lambda i: (0, 0)),
                pl.BlockSpec((CO, 1), lambda i: (0, 0)),
            ],
            out_specs=pl.BlockSpec((1, CO, NBLK_C), lambda i: (0, 0, i)),
            out_shape=jax.ShapeDtypeStruct((1, CO, N), f32),
        )(pre_b, A_b, X1, Wfc, bfc2)
        s1_parts.append(s1_b)

    S1 = jnp.concatenate(s1_parts, axis=0)
    return (P1, S1)
